# baseline (speedup 1.0000x reference)
import jax
import jax.numpy as jnp
from jax.experimental import pallas as pl
from jax.experimental.pallas import tpu as pltpu

_ROWS = 65536
_D = 6 * 7 * 8


def _dma_kernel(x_hbm, o_hbm, t_vmem, sem_big, sem_t):
    flat = (jax.lax.broadcasted_iota(jnp.int32, (2, _D), 0) * _D
            + jax.lax.broadcasted_iota(jnp.int32, (2, _D), 1))
    t_vmem[...] = flat.astype(jnp.float32)

    big = pltpu.make_async_copy(x_hbm, o_hbm, sem_big)
    big.start()
    big.wait()
    small = pltpu.make_async_copy(t_vmem, o_hbm.at[pl.ds(0, 2)], sem_t)
    small.start()
    small.wait()


def kernel(x):
    xf = x.reshape(_ROWS, _D)
    out = pl.pallas_call(
        _dma_kernel,
        in_specs=[pl.BlockSpec(memory_space=pltpu.MemorySpace.HBM)],
        out_specs=pl.BlockSpec(memory_space=pltpu.MemorySpace.HBM),
        out_shape=jax.ShapeDtypeStruct((_ROWS, _D), jnp.float32),
        scratch_shapes=[
            pltpu.VMEM((2, _D), jnp.float32),
            pltpu.SemaphoreType.DMA,
            pltpu.SemaphoreType.DMA,
        ],
    )(xf)
    return out.reshape(x.shape)


# speedup vs baseline: 1.0009x; 1.0009x over previous
import jax
import jax.numpy as jnp
from jax.experimental import pallas as pl
from jax.experimental.pallas import tpu as pltpu

_ROWS = 65536
_D = 6 * 7 * 8


_NCHUNK = 16
_CHUNK = _ROWS // _NCHUNK


def _dma_kernel(x_hbm, o_hbm, t_vmem, sem_big, sem_t):
    flat = (jax.lax.broadcasted_iota(jnp.int32, (2, _D), 0) * _D
            + jax.lax.broadcasted_iota(jnp.int32, (2, _D), 1))
    t_vmem[...] = flat.astype(jnp.float32)

    copies = [
        pltpu.make_async_copy(
            x_hbm.at[pl.ds(c * _CHUNK, _CHUNK)],
            o_hbm.at[pl.ds(c * _CHUNK, _CHUNK)],
            sem_big.at[c],
        )
        for c in range(_NCHUNK)
    ]
    for cp in copies:
        cp.start()
    for cp in copies:
        cp.wait()
    small = pltpu.make_async_copy(t_vmem, o_hbm.at[pl.ds(0, 2)], sem_t)
    small.start()
    small.wait()


def kernel(x):
    xf = x.reshape(_ROWS, _D)
    out = pl.pallas_call(
        _dma_kernel,
        in_specs=[pl.BlockSpec(memory_space=pltpu.MemorySpace.HBM)],
        out_specs=pl.BlockSpec(memory_space=pltpu.MemorySpace.HBM),
        out_shape=jax.ShapeDtypeStruct((_ROWS, _D), jnp.float32),
        scratch_shapes=[
            pltpu.VMEM((2, _D), jnp.float32),
            pltpu.SemaphoreType.DMA((_NCHUNK,)),
            pltpu.SemaphoreType.DMA,
        ],
    )(xf)
    return out.reshape(x.shape)


# speedup vs baseline: 13.8516x; 13.8398x over previous
import jax
import jax.numpy as jnp
from jax.experimental import pallas as pl

_ROWS = 65536
_D = 6 * 7 * 8
_BLOCK = 8192


def _copy_kernel(x_ref, o_ref):
    o_ref[...] = x_ref[...]

    @pl.when(pl.program_id(0) == 0)
    def _():
        flat = (jax.lax.broadcasted_iota(jnp.int32, (2, _D), 0) * _D
                + jax.lax.broadcasted_iota(jnp.int32, (2, _D), 1))
        o_ref[0:2, :] = flat.astype(jnp.float32)


def kernel(x):
    xf = x.reshape(_ROWS, _D)
    out = pl.pallas_call(
        _copy_kernel,
        grid=(_ROWS // _BLOCK,),
        in_specs=[pl.BlockSpec((_BLOCK, _D), lambda i: (i, 0))],
        out_specs=pl.BlockSpec((_BLOCK, _D), lambda i: (i, 0)),
        out_shape=jax.ShapeDtypeStruct((_ROWS, _D), jnp.float32),
    )(xf)
    return out.reshape(x.shape)
